# Initial kernel scaffold; baseline (speedup 1.0000x reference)
import jax
import jax.numpy as jnp
from jax.experimental import pallas as pl


def kernel(w, z_embed, f, z, xyz, total_charge, num_atoms, mol_nbrs):
    raise NotImplementedError("write your pallas kernel here")


# speedup vs baseline: 1.0014x; 1.0014x over previous
import functools

import jax
import jax.numpy as jnp
from jax import lax
from jax.experimental import pallas as pl
from jax.experimental.pallas import tpu as pltpu

_EPS = 1e-15
_BOHR2 = 0.529177 * 0.529177
_KE = 332.0637


def _charge_kernel(f_ref, w_ref, qz_ref, q_ref):
    acc = jnp.dot(f_ref[...], w_ref[...], preferred_element_type=jnp.float32)
    q_ref[...] = acc[:, 0:1] + qz_ref[...]


def _energy_kernel(r2_ref, qq_ref, mid_ref, out_ref, *, r_on, r_off, n_mols):
    @pl.when(pl.program_id(1) == 0)
    def _init():
        out_ref[...] = jnp.zeros_like(out_ref)

    r2 = r2_ref[...]
    qq = qq_ref[...]
    mid = mid_ref[...]

    inv_r = lax.rsqrt(r2)
    r = r2 * inv_r
    inv_width = 1.0 / (r_off - r_on)
    x = (r - r_on) * inv_width
    y = 1.0 - x
    inside = jnp.logical_and(x > 0, y > 0)
    den = jnp.where(inside, x * y, 1.0)
    earg = jnp.clip((x - y) * pl.reciprocal(den, approx=True), -60.0, 34.0)
    sig = pl.reciprocal(1.0 + jnp.exp(earg), approx=True)
    sig = jnp.where(earg >= 34.0, 0.0, sig)
    fs = jnp.where(x <= 0, 1.0, jnp.where(y <= 0, 0.0, sig))
    pw = (_KE * qq) * (fs * lax.rsqrt(r2 + _BOHR2) + (1.0 - fs) * inv_r)

    tr = pw.shape[0]
    pw3 = pw.reshape(tr // 8, 8, 128)
    mid3 = mid.reshape(tr // 8, 8, 128)
    for m in range(n_mols):
        part = jnp.sum(jnp.where(mid3 == m, pw3, 0.0), axis=0)
        out_ref[0, pl.ds(8 * m, 8), :] += part


def kernel(w, z_embed, f, z, xyz, total_charge, num_atoms, mol_nbrs):
    r_cut = 5.0
    r_on, r_off = r_cut / 4.0, 3.0 * r_cut / 4.0
    n_atoms, feat_dim = f.shape
    n_mols = num_atoms.shape[0]
    n_pairs = mol_nbrs.shape[0]

    atom_tile = 2048
    n_pad_atoms = -(-n_atoms // atom_tile) * atom_tile
    w8 = jnp.pad(w.astype(jnp.float32), ((0, 0), (0, 7)))
    qz = jnp.take(z_embed, z, axis=0).astype(jnp.float32)
    f_pad = jnp.pad(f.astype(jnp.float32), ((0, n_pad_atoms - n_atoms), (0, 0)))
    qz_pad = jnp.pad(qz, ((0, n_pad_atoms - n_atoms), (0, 0)))

    charge = pl.pallas_call(
        _charge_kernel,
        out_shape=jax.ShapeDtypeStruct((n_pad_atoms, 1), jnp.float32),
        grid=(n_pad_atoms // atom_tile,),
        in_specs=[
            pl.BlockSpec((atom_tile, feat_dim), lambda i: (i, 0)),
            pl.BlockSpec((feat_dim, 8), lambda i: (0, 0)),
            pl.BlockSpec((atom_tile, 1), lambda i: (i, 0)),
        ],
        out_specs=pl.BlockSpec((atom_tile, 1), lambda i: (i, 0)),
        compiler_params=pltpu.CompilerParams(dimension_semantics=("parallel",)),
    )(f_pad, w8, qz_pad)[:n_atoms]

    mol_of_atom = jnp.repeat(jnp.arange(n_mols, dtype=jnp.int32), num_atoms,
                             total_repeat_length=n_atoms)
    msum = jax.ops.segment_sum(charge[:, 0], mol_of_atom, num_segments=n_mols,
                               indices_are_sorted=True)
    corr = (total_charge.astype(jnp.float32) - msum) / num_atoms.astype(jnp.float32)
    q = charge + jnp.take(corr, mol_of_atom)[:, None]

    tile_pairs = 16384
    num_chunks = 2
    tile_rows = tile_pairs // 128
    n_tiles = max(1, -(-n_pairs // tile_pairs))
    tpc = -(-n_tiles // num_chunks)
    p_pad = num_chunks * tpc * tile_pairs
    pad = p_pad - n_pairs
    p_rows = p_pad // 128

    idx_i = jnp.pad(mol_nbrs[:, 0], (0, pad))
    idx_j = jnp.pad(mol_nbrs[:, 1], (0, pad))
    xyz_f = xyz.astype(jnp.float32)
    d0 = jnp.take(xyz_f[:, 0], idx_i) - jnp.take(xyz_f[:, 0], idx_j)
    d1 = jnp.take(xyz_f[:, 1], idx_i) - jnp.take(xyz_f[:, 1], idx_j)
    d2 = jnp.take(xyz_f[:, 2], idx_i) - jnp.take(xyz_f[:, 2], idx_j)
    r2 = (d0 * d0 + d1 * d1 + d2 * d2 + 3.0 * _EPS).reshape(p_rows, 128)
    q_flat = q[:, 0]
    qq = (jnp.take(q_flat, idx_i) * jnp.take(q_flat, idx_j)).reshape(p_rows, 128)
    mid = jnp.pad(jnp.take(mol_of_atom, mol_nbrs[:, 0]), (0, pad),
                  constant_values=n_mols).astype(jnp.int32).reshape(p_rows, 128)

    pair_spec = pl.BlockSpec((tile_rows, 128), lambda c, t: (c * tpc + t, 0))
    out = pl.pallas_call(
        functools.partial(_energy_kernel, r_on=r_on, r_off=r_off, n_mols=n_mols),
        out_shape=jax.ShapeDtypeStruct((num_chunks, 8 * n_mols, 128), jnp.float32),
        grid=(num_chunks, tpc),
        in_specs=[pair_spec] * 3,
        out_specs=pl.BlockSpec((1, 8 * n_mols, 128), lambda c, t: (c, 0, 0)),
        compiler_params=pltpu.CompilerParams(
            dimension_semantics=("parallel", "arbitrary"),
            vmem_limit_bytes=48 * 1024 * 1024),
    )(r2, qq, mid)

    energy = out.reshape(num_chunks, n_mols, 8, 128).sum(axis=(0, 2, 3))[:, None]
    return energy, q
